# baseline (speedup 1.0000x reference)
import jax
import jax.numpy as jnp
from jax import lax
from jax.experimental import pallas as pl
from jax.experimental.pallas import tpu as pltpu
from jax.experimental.pallas import tpu_sc as plsc

N_NODES = 10000
N_EDGES = 320000
BN_EPS = 1e-5

NC = 2
NS = 16
NW = NC * NS
CHUNK = 128
CH_PER_TILE = 80
EDGES_PER_TILE = CH_PER_TILE * CHUNK
E_PAD = NW * EDGES_PER_TILE
N_ACC = 10112
RPT = N_ACC // NS
CHT = 160


def _mesh():
    return plsc.VectorSubcoreMesh(core_axis_name="c", subcore_axis_name="s")


def _sc_degree(dst2d, ones, zeros1):
    def body(dst_hbm, ones_hbm, zeros_hbm, out_hbm, dst_v, ones_v, zbuf_v,
             acc_sh, sem):
        cid = lax.axis_index("c")
        sid = lax.axis_index("s")
        wid = cid * NS + sid
        pltpu.sync_copy(zeros_hbm, zbuf_v)
        pltpu.sync_copy(zbuf_v, acc_sh.at[pl.ds(sid * RPT, RPT)])
        pltpu.sync_copy(dst_hbm.at[pl.ds(wid * CH_PER_TILE, CH_PER_TILE)], dst_v)
        pltpu.sync_copy(ones_hbm, ones_v)
        plsc.subcore_barrier()

        def step(j, carry):
            pltpu.sync_copy(ones_v, acc_sh.at[dst_v.at[j]], add=True)
            return carry

        lax.fori_loop(0, CH_PER_TILE, step, 0)
        plsc.subcore_barrier()
        pltpu.sync_copy(acc_sh.at[pl.ds(sid * RPT, RPT)], zbuf_v)
        pltpu.sync_copy(zbuf_v, out_hbm.at[pl.ds(cid * N_ACC + sid * RPT, RPT)])

    k = pl.kernel(
        body,
        out_type=jax.ShapeDtypeStruct((NC * N_ACC,), jnp.float32),
        mesh=_mesh(),
        scratch_types=[
            pltpu.VMEM((CH_PER_TILE, CHUNK), jnp.int32),
            pltpu.VMEM((CHUNK,), jnp.float32),
            pltpu.VMEM((RPT,), jnp.float32),
            pltpu.VMEM_SHARED((N_ACC,), jnp.float32),
            pltpu.SemaphoreType.DMA,
        ],
    )
    return k(dst2d, ones, zeros1)


def _sc_propagate(src2d, dst2d, table, d):
    K = 4
    LA = 2
    chmax = CHT
    stg = 79
    nstg = RPT // stg

    def body(src_hbm, dst_hbm, table_hbm, out_hbm,
             src_v, dst_v, rows_v, acc_sh, *sems):
        semg = sems[:K]
        sems_ = sems[K:]
        cid = lax.axis_index("c")
        sid = lax.axis_index("s")

        def buf(b):
            return rows_v.at[pl.ds(b * CHUNK, CHUNK)]

        def prologue():
            def zrow(r, c):
                for kk in range(d // 16):
                    rows_v[r, pl.ds(kk * 16, 16)] = jnp.zeros((16,),
                                                              jnp.float32)
                return c

            lax.fori_loop(0, stg, zrow, 0)
            stag = rows_v.at[pl.ds(0, stg)]
            for cc in range(nstg):
                pltpu.sync_copy(stag,
                                acc_sh.at[pl.ds(sid * RPT + cc * stg, stg)])

        def run(ch, base):
            pltpu.async_copy(src_hbm.at[pl.ds(base, ch)],
                             src_v.at[pl.ds(0, ch)], semg[0])
            pltpu.async_copy(dst_hbm.at[pl.ds(base, ch)],
                             dst_v.at[pl.ds(0, ch)], semg[1])
            pltpu.make_async_copy(src_hbm.at[pl.ds(base, ch)],
                                  src_v.at[pl.ds(0, ch)], semg[0]).wait()
            pltpu.make_async_copy(dst_hbm.at[pl.ds(base, ch)],
                                  dst_v.at[pl.ds(0, ch)], semg[1]).wait()
            ngroups = ch // K

            for b in range(LA):
                pltpu.async_copy(table_hbm.at[src_v.at[b]], buf(b), semg[b])

            def pos_step(j, pos, first_group):
                pltpu.make_async_copy(table_hbm.at[src_v.at[j]], buf(pos),
                                      semg[pos]).wait()
                pltpu.async_copy(buf(pos), acc_sh.at[dst_v.at[j]], sems_[pos],
                                 add=True)
                jn = j + LA
                bn = (pos + LA) % K
                if first_group:
                    if jn < K:
                        pltpu.async_copy(table_hbm.at[src_v.at[jn]], buf(bn),
                                         semg[bn])
                    else:
                        pltpu.make_async_copy(buf(bn),
                                              acc_sh.at[dst_v.at[jn]],
                                              sems_[bn]).wait()
                        pltpu.async_copy(table_hbm.at[src_v.at[jn]], buf(bn),
                                         semg[bn])
                else:
                    @pl.when(jn < ch)
                    def _():
                        pltpu.make_async_copy(buf(bn),
                                              acc_sh.at[dst_v.at[jn]],
                                              sems_[bn]).wait()
                        pltpu.async_copy(table_hbm.at[src_v.at[jn]], buf(bn),
                                         semg[bn])

            for pos in range(K):
                pos_step(pos, pos, True)

            def step(g, carry):
                for pos in range(K):
                    pos_step(g * K + pos, pos, False)
                return carry

            lax.fori_loop(1, ngroups, step, 0)
            for pos in range(K):
                j = (ngroups - 1) * K + pos
                pltpu.make_async_copy(buf(pos), acc_sh.at[dst_v.at[j]],
                                      sems_[pos]).wait()

        @pl.when(cid == 0)
        def _():
            prologue()
            run(CHT, sid * CHT)
            plsc.subcore_barrier()
            def oslot(c):
                return rows_v.at[pl.ds((c % K) * CHUNK, stg)]

            def osrc(c):
                return acc_sh.at[pl.ds(sid * RPT + c * stg, stg)]

            def odst(c):
                return out_hbm.at[pl.ds(sid * RPT + c * stg, stg)]

            for c in range(nstg):
                if c >= K:
                    pltpu.make_async_copy(oslot(c - K), odst(c - K),
                                          sems_[c % K]).wait()
                pltpu.sync_copy(osrc(c), oslot(c))
                pltpu.async_copy(oslot(c), odst(c), sems_[c % K])
            for c in range(nstg - K, nstg):
                pltpu.make_async_copy(oslot(c), odst(c), sems_[c % K]).wait()

    k = pl.kernel(
        body,
        out_type=jax.ShapeDtypeStruct((N_ACC, d), jnp.float32),
        mesh=_mesh(),
        compiler_params=pltpu.CompilerParams(use_tc_tiling_on_sc=False),
        scratch_types=[
            pltpu.VMEM((chmax, CHUNK), jnp.int32),
            pltpu.VMEM((chmax, CHUNK), jnp.int32),
            pltpu.VMEM((K * CHUNK, d), jnp.float32),
            pltpu.VMEM_SHARED((N_ACC, d), jnp.float32),
        ] + [pltpu.SemaphoreType.DMA] * (2 * K),
    )
    return k(src2d, dst2d, table)


BLK = 1000


def _tc1_body(x_ref, w1_ref, hist_ref, m1a_ref, m1b_ref, st1_ref, dis_ref):
    deg = hist_ref[0] + hist_ref[1] + 1.0
    dis = lax.rsqrt(deg)
    inv = 1.0 / deg
    h = jnp.dot(x_ref[...], w1_ref[...], preferred_element_type=jnp.float32)
    m1 = h * dis
    m1a_ref[...] = m1[:, :64]
    m1b_ref[...] = m1[:, 64:]
    st1_ref[...] = h * inv
    dis_ref[...] = dis


def _tc1(x, w1, hist):
    hist3 = hist.reshape(NC, N_ACC, 1)
    return pl.pallas_call(
        _tc1_body,
        grid=(N_NODES // BLK,),
        in_specs=[
            pl.BlockSpec((BLK, 128), lambda i: (i, 0)),
            pl.BlockSpec((128, 128), lambda i: (0, 0)),
            pl.BlockSpec((NC, BLK, 1), lambda i: (0, i, 0)),
        ],
        out_specs=[
            pl.BlockSpec((BLK, 64), lambda i: (i, 0)),
            pl.BlockSpec((BLK, 64), lambda i: (i, 0)),
            pl.BlockSpec((BLK, 128), lambda i: (i, 0)),
            pl.BlockSpec((BLK, 1), lambda i: (i, 0)),
        ],
        out_shape=[
            jax.ShapeDtypeStruct((N_NODES, 64), jnp.float32),
            jax.ShapeDtypeStruct((N_NODES, 64), jnp.float32),
            jax.ShapeDtypeStruct((N_NODES, 128), jnp.float32),
            jax.ShapeDtypeStruct((N_NODES, 1), jnp.float32),
        ],
    )(x, w1, hist3)


def _tc2_body(pa_ref, pb_ref, st1_ref, dis_ref, g_ref, bb_ref, w2_ref,
              m2_ref, st2_ref):
    dis = dis_ref[...]
    scat = jnp.concatenate([pa_ref[...], pb_ref[...]], axis=1)
    prop = scat * dis + st1_ref[...]
    t = jnp.maximum(prop * g_ref[...] + bb_ref[...], 0.0)
    h2 = jnp.dot(t, w2_ref[...], preferred_element_type=jnp.float32)
    m2_ref[...] = h2 * dis
    st2_ref[...] = h2 * (dis * dis)


def _tc2(p1a, p1b, st1, dis, gscale, bshift, w2):
    return pl.pallas_call(
        _tc2_body,
        grid=(N_NODES // BLK,),
        in_specs=[
            pl.BlockSpec((BLK, 64), lambda i: (i, 0)),
            pl.BlockSpec((BLK, 64), lambda i: (i, 0)),
            pl.BlockSpec((BLK, 128), lambda i: (i, 0)),
            pl.BlockSpec((BLK, 1), lambda i: (i, 0)),
            pl.BlockSpec((1, 128), lambda i: (0, 0)),
            pl.BlockSpec((1, 128), lambda i: (0, 0)),
            pl.BlockSpec((128, 64), lambda i: (0, 0)),
        ],
        out_specs=[
            pl.BlockSpec((BLK, 64), lambda i: (i, 0)),
            pl.BlockSpec((BLK, 64), lambda i: (i, 0)),
        ],
        out_shape=[
            jax.ShapeDtypeStruct((N_NODES, 64), jnp.float32),
            jax.ShapeDtypeStruct((N_NODES, 64), jnp.float32),
        ],
    )(p1a, p1b, st1, dis, gscale, bshift, w2)


def _tc3_body(p_ref, st2_ref, dis_ref, b2_ref, w3_ref, m3_ref, st3_ref):
    dis = dis_ref[...]
    out2 = jnp.maximum(
        p_ref[...] * dis + st2_ref[...] + b2_ref[...], 0.0)
    h3 = jnp.dot(out2, w3_ref[...], preferred_element_type=jnp.float32)
    m3_ref[...] = h3 * dis
    st3_ref[...] = h3[:, :2] * (dis * dis)


def _tc3(p2, st2, dis, b2, w3):
    return pl.pallas_call(
        _tc3_body,
        grid=(N_NODES // BLK,),
        in_specs=[
            pl.BlockSpec((BLK, 64), lambda i: (i, 0)),
            pl.BlockSpec((BLK, 64), lambda i: (i, 0)),
            pl.BlockSpec((BLK, 1), lambda i: (i, 0)),
            pl.BlockSpec((1, 64), lambda i: (0, 0)),
            pl.BlockSpec((64, 16), lambda i: (0, 0)),
        ],
        out_specs=[
            pl.BlockSpec((BLK, 16), lambda i: (i, 0)),
            pl.BlockSpec((BLK, 2), lambda i: (i, 0)),
        ],
        out_shape=[
            jax.ShapeDtypeStruct((N_NODES, 16), jnp.float32),
            jax.ShapeDtypeStruct((N_NODES, 2), jnp.float32),
        ],
    )(p2, st2, dis, b2, w3)


def _tc4_body(p_ref, st3_ref, dis_ref, b3_ref, out_ref):
    scat = p_ref[:, :2]
    out_ref[...] = scat * dis_ref[...] + st3_ref[...] + b3_ref[...]


def _tc4(p3, st3, dis, b3):
    return pl.pallas_call(
        _tc4_body,
        grid=(N_NODES // BLK,),
        in_specs=[
            pl.BlockSpec((BLK, 16), lambda i: (i, 0)),
            pl.BlockSpec((BLK, 2), lambda i: (i, 0)),
            pl.BlockSpec((BLK, 1), lambda i: (i, 0)),
            pl.BlockSpec((1, 2), lambda i: (0, 0)),
        ],
        out_specs=pl.BlockSpec((BLK, 2), lambda i: (i, 0)),
        out_shape=jax.ShapeDtypeStruct((N_NODES, 2), jnp.float32),
    )(p3, st3, dis, b3)


def kernel(x, edge_index, W1, b1, gamma, beta, W2, b2, W3, b3):
    ei = edge_index.astype(jnp.int32)
    pad = E_PAD - N_EDGES
    src = jnp.concatenate([ei[0], jnp.zeros((pad,), jnp.int32)])
    dst = jnp.concatenate([ei[1], jnp.full((pad,), N_NODES, jnp.int32)])
    src2d = src.reshape(E_PAD // CHUNK, CHUNK)
    dst2d = dst.reshape(E_PAD // CHUNK, CHUNK)

    zeros1 = jnp.zeros((RPT,), jnp.float32)
    ones = jnp.ones((CHUNK,), jnp.float32)

    hist = _sc_degree(dst2d, ones, zeros1)
    m1a, m1b, st1, dis = _tc1(x, W1, hist)
    p1a = _sc_propagate(src2d, dst2d, m1a, 64)
    p1b = _sc_propagate(src2d, dst2d, m1b, 64)

    c = (1.0 + BN_EPS) ** -0.5
    gscale = (gamma * c).reshape(1, 128)
    bshift = (beta + b1 * gamma * c).reshape(1, 128)
    m2, st2 = _tc2(p1a, p1b, st1, dis, gscale, bshift, W2)
    p2 = _sc_propagate(src2d, dst2d, m2, 64)

    w3p = jnp.concatenate([W3, jnp.zeros((64, 14), jnp.float32)], axis=1)
    m3, st3 = _tc3(p2, st2, dis, b2.reshape(1, 64), w3p)
    p3 = _sc_propagate(src2d, dst2d, m3, 16)

    return _tc4(p3, st3, dis, b3.reshape(1, 2))


# speedup vs baseline: 1.1818x; 1.1818x over previous
import jax
import jax.numpy as jnp
from jax import lax
from jax.experimental import pallas as pl
from jax.experimental.pallas import tpu as pltpu
from jax.experimental.pallas import tpu_sc as plsc

N_NODES = 10000
N_EDGES = 320000
BN_EPS = 1e-5

NC = 2
NS = 16
NW = NC * NS
CHUNK = 128
CH_PER_TILE = 80
EDGES_PER_TILE = CH_PER_TILE * CHUNK
E_PAD = NW * EDGES_PER_TILE
N_ACC = 10112
RPT = N_ACC // NS
CH0 = 132
CH1 = 28


def _mesh():
    return plsc.VectorSubcoreMesh(core_axis_name="c", subcore_axis_name="s")


def _sc_degree(dst2d, ones, zeros1):
    def body(dst_hbm, ones_hbm, zeros_hbm, out_hbm, dst_v, ones_v, zbuf_v,
             acc_sh, sem):
        cid = lax.axis_index("c")
        sid = lax.axis_index("s")
        wid = cid * NS + sid
        pltpu.sync_copy(zeros_hbm, zbuf_v)
        pltpu.sync_copy(zbuf_v, acc_sh.at[pl.ds(sid * RPT, RPT)])
        pltpu.sync_copy(dst_hbm.at[pl.ds(wid * CH_PER_TILE, CH_PER_TILE)], dst_v)
        pltpu.sync_copy(ones_hbm, ones_v)
        plsc.subcore_barrier()

        def step(j, carry):
            pltpu.sync_copy(ones_v, acc_sh.at[dst_v.at[j]], add=True)
            return carry

        lax.fori_loop(0, CH_PER_TILE, step, 0)
        plsc.subcore_barrier()
        pltpu.sync_copy(acc_sh.at[pl.ds(sid * RPT, RPT)], zbuf_v)
        pltpu.sync_copy(zbuf_v, out_hbm.at[pl.ds(cid * N_ACC + sid * RPT, RPT)])

    k = pl.kernel(
        body,
        out_type=jax.ShapeDtypeStruct((NC * N_ACC,), jnp.float32),
        mesh=_mesh(),
        scratch_types=[
            pltpu.VMEM((CH_PER_TILE, CHUNK), jnp.int32),
            pltpu.VMEM((CHUNK,), jnp.float32),
            pltpu.VMEM((RPT,), jnp.float32),
            pltpu.VMEM_SHARED((N_ACC,), jnp.float32),
            pltpu.SemaphoreType.DMA,
        ],
    )
    return k(dst2d, ones, zeros1)


def _sc_propagate(src2d, dst2d, table, d, ch0=CH_PER_TILE,
                  ch1=CH_PER_TILE):
    K = 4
    LA = 2
    chmax = max(ch0, ch1)
    stg = 79
    nstg = RPT // stg

    def body(src_hbm, dst_hbm, table_hbm, out_hbm,
             src_v, dst_v, rows_v, acc_sh, *sems):
        semg = sems[:K]
        sems_ = sems[K:]
        cid = lax.axis_index("c")
        sid = lax.axis_index("s")

        def buf(b):
            return rows_v.at[pl.ds(b * CHUNK, CHUNK)]

        def zrow(r, c):
            for kk in range(d // 16):
                rows_v[r, pl.ds(kk * 16, 16)] = jnp.zeros((16,), jnp.float32)
            return c

        lax.fori_loop(0, stg, zrow, 0)
        stag = rows_v.at[pl.ds(0, stg)]
        for cc in range(nstg):
            pltpu.sync_copy(stag, acc_sh.at[pl.ds(sid * RPT + cc * stg, stg)])

        def run(ch, base):
            pltpu.async_copy(src_hbm.at[pl.ds(base, ch)],
                             src_v.at[pl.ds(0, ch)], semg[0])
            pltpu.async_copy(dst_hbm.at[pl.ds(base, ch)],
                             dst_v.at[pl.ds(0, ch)], semg[1])
            pltpu.make_async_copy(src_hbm.at[pl.ds(base, ch)],
                                  src_v.at[pl.ds(0, ch)], semg[0]).wait()
            pltpu.make_async_copy(dst_hbm.at[pl.ds(base, ch)],
                                  dst_v.at[pl.ds(0, ch)], semg[1]).wait()
            ngroups = ch // K

            for b in range(LA):
                pltpu.async_copy(table_hbm.at[src_v.at[b]], buf(b), semg[b])

            def pos_step(j, pos, first_group):
                pltpu.make_async_copy(table_hbm.at[src_v.at[j]], buf(pos),
                                      semg[pos]).wait()
                pltpu.async_copy(buf(pos), acc_sh.at[dst_v.at[j]], sems_[pos],
                                 add=True)
                jn = j + LA
                bn = (pos + LA) % K
                if first_group:
                    if jn < K:
                        pltpu.async_copy(table_hbm.at[src_v.at[jn]], buf(bn),
                                         semg[bn])
                    else:
                        pltpu.make_async_copy(buf(bn),
                                              acc_sh.at[dst_v.at[jn]],
                                              sems_[bn]).wait()
                        pltpu.async_copy(table_hbm.at[src_v.at[jn]], buf(bn),
                                         semg[bn])
                else:
                    @pl.when(jn < ch)
                    def _():
                        pltpu.make_async_copy(buf(bn),
                                              acc_sh.at[dst_v.at[jn]],
                                              sems_[bn]).wait()
                        pltpu.async_copy(table_hbm.at[src_v.at[jn]], buf(bn),
                                         semg[bn])

            for pos in range(K):
                pos_step(pos, pos, True)

            def step(g, carry):
                for pos in range(K):
                    pos_step(g * K + pos, pos, False)
                return carry

            lax.fori_loop(1, ngroups, step, 0)
            for pos in range(K):
                j = (ngroups - 1) * K + pos
                pltpu.make_async_copy(buf(pos), acc_sh.at[dst_v.at[j]],
                                      sems_[pos]).wait()

        @pl.when(cid == 0)
        def _():
            run(ch0, sid * ch0)

        @pl.when(cid == 1)
        def _():
            run(ch1, NS * ch0 + sid * ch1)

        plsc.subcore_barrier()
        def oslot(c):
            return rows_v.at[pl.ds((c % K) * CHUNK, stg)]

        def osrc(c):
            return acc_sh.at[pl.ds(sid * RPT + c * stg, stg)]

        def odst(c):
            return out_hbm.at[cid, pl.ds(sid * RPT + c * stg, stg)]

        for c in range(nstg):
            if c >= K:
                pltpu.make_async_copy(oslot(c - K), odst(c - K),
                                      sems_[c % K]).wait()
            pltpu.sync_copy(osrc(c), oslot(c))
            pltpu.async_copy(oslot(c), odst(c), sems_[c % K])
        for c in range(nstg - K, nstg):
            pltpu.make_async_copy(oslot(c), odst(c), sems_[c % K]).wait()

    k = pl.kernel(
        body,
        out_type=jax.ShapeDtypeStruct((NC, N_ACC, d), jnp.float32),
        mesh=_mesh(),
        compiler_params=pltpu.CompilerParams(use_tc_tiling_on_sc=False),
        scratch_types=[
            pltpu.VMEM((chmax, CHUNK), jnp.int32),
            pltpu.VMEM((chmax, CHUNK), jnp.int32),
            pltpu.VMEM((K * CHUNK, d), jnp.float32),
            pltpu.VMEM_SHARED((N_ACC, d), jnp.float32),
        ] + [pltpu.SemaphoreType.DMA] * (2 * K),
    )
    return k(src2d, dst2d, table)


BLK = 1000


def _tc1_body(x_ref, w1_ref, hist_ref, m1a_ref, m1b_ref, st1_ref, dis_ref):
    deg = hist_ref[0] + hist_ref[1] + 1.0
    dis = lax.rsqrt(deg)
    inv = 1.0 / deg
    h = jnp.dot(x_ref[...], w1_ref[...], preferred_element_type=jnp.float32)
    m1 = h * dis
    m1a_ref[...] = m1[:, :64]
    m1b_ref[...] = m1[:, 64:]
    st1_ref[...] = h * inv
    dis_ref[...] = dis


def _tc1(x, w1, hist):
    hist3 = hist.reshape(NC, N_ACC, 1)
    return pl.pallas_call(
        _tc1_body,
        grid=(N_NODES // BLK,),
        in_specs=[
            pl.BlockSpec((BLK, 128), lambda i: (i, 0)),
            pl.BlockSpec((128, 128), lambda i: (0, 0)),
            pl.BlockSpec((NC, BLK, 1), lambda i: (0, i, 0)),
        ],
        out_specs=[
            pl.BlockSpec((BLK, 64), lambda i: (i, 0)),
            pl.BlockSpec((BLK, 64), lambda i: (i, 0)),
            pl.BlockSpec((BLK, 128), lambda i: (i, 0)),
            pl.BlockSpec((BLK, 1), lambda i: (i, 0)),
        ],
        out_shape=[
            jax.ShapeDtypeStruct((N_NODES, 64), jnp.float32),
            jax.ShapeDtypeStruct((N_NODES, 64), jnp.float32),
            jax.ShapeDtypeStruct((N_NODES, 128), jnp.float32),
            jax.ShapeDtypeStruct((N_NODES, 1), jnp.float32),
        ],
    )(x, w1, hist3)


def _tc2_body(pa_ref, pb_ref, st1_ref, dis_ref, g_ref, bb_ref, w2_ref,
              m2_ref, st2_ref):
    dis = dis_ref[...]
    scat = jnp.concatenate([pa_ref[0] + pa_ref[1], pb_ref[0] + pb_ref[1]],
                           axis=1)
    prop = scat * dis + st1_ref[...]
    t = jnp.maximum(prop * g_ref[...] + bb_ref[...], 0.0)
    h2 = jnp.dot(t, w2_ref[...], preferred_element_type=jnp.float32)
    m2_ref[...] = h2 * dis
    st2_ref[...] = h2 * (dis * dis)


def _tc2(p1a, p1b, st1, dis, gscale, bshift, w2):
    return pl.pallas_call(
        _tc2_body,
        grid=(N_NODES // BLK,),
        in_specs=[
            pl.BlockSpec((NC, BLK, 64), lambda i: (0, i, 0)),
            pl.BlockSpec((NC, BLK, 64), lambda i: (0, i, 0)),
            pl.BlockSpec((BLK, 128), lambda i: (i, 0)),
            pl.BlockSpec((BLK, 1), lambda i: (i, 0)),
            pl.BlockSpec((1, 128), lambda i: (0, 0)),
            pl.BlockSpec((1, 128), lambda i: (0, 0)),
            pl.BlockSpec((128, 64), lambda i: (0, 0)),
        ],
        out_specs=[
            pl.BlockSpec((BLK, 64), lambda i: (i, 0)),
            pl.BlockSpec((BLK, 64), lambda i: (i, 0)),
        ],
        out_shape=[
            jax.ShapeDtypeStruct((N_NODES, 64), jnp.float32),
            jax.ShapeDtypeStruct((N_NODES, 64), jnp.float32),
        ],
    )(p1a, p1b, st1, dis, gscale, bshift, w2)


def _tc3_body(p_ref, st2_ref, dis_ref, b2_ref, w3_ref, m3_ref, st3_ref):
    dis = dis_ref[...]
    out2 = jnp.maximum(
        (p_ref[0] + p_ref[1]) * dis + st2_ref[...] + b2_ref[...], 0.0)
    h3 = jnp.dot(out2, w3_ref[...], preferred_element_type=jnp.float32)
    m3_ref[...] = h3 * dis
    st3_ref[...] = h3[:, :2] * (dis * dis)


def _tc3(p2, st2, dis, b2, w3):
    return pl.pallas_call(
        _tc3_body,
        grid=(N_NODES // BLK,),
        in_specs=[
            pl.BlockSpec((NC, BLK, 64), lambda i: (0, i, 0)),
            pl.BlockSpec((BLK, 64), lambda i: (i, 0)),
            pl.BlockSpec((BLK, 1), lambda i: (i, 0)),
            pl.BlockSpec((1, 64), lambda i: (0, 0)),
            pl.BlockSpec((64, 16), lambda i: (0, 0)),
        ],
        out_specs=[
            pl.BlockSpec((BLK, 16), lambda i: (i, 0)),
            pl.BlockSpec((BLK, 2), lambda i: (i, 0)),
        ],
        out_shape=[
            jax.ShapeDtypeStruct((N_NODES, 16), jnp.float32),
            jax.ShapeDtypeStruct((N_NODES, 2), jnp.float32),
        ],
    )(p2, st2, dis, b2, w3)


def _tc4_body(p_ref, st3_ref, dis_ref, b3_ref, out_ref):
    scat = (p_ref[0] + p_ref[1])[:, :2]
    out_ref[...] = scat * dis_ref[...] + st3_ref[...] + b3_ref[...]


def _tc4(p3, st3, dis, b3):
    return pl.pallas_call(
        _tc4_body,
        grid=(N_NODES // BLK,),
        in_specs=[
            pl.BlockSpec((NC, BLK, 16), lambda i: (0, i, 0)),
            pl.BlockSpec((BLK, 2), lambda i: (i, 0)),
            pl.BlockSpec((BLK, 1), lambda i: (i, 0)),
            pl.BlockSpec((1, 2), lambda i: (0, 0)),
        ],
        out_specs=pl.BlockSpec((BLK, 2), lambda i: (i, 0)),
        out_shape=jax.ShapeDtypeStruct((N_NODES, 2), jnp.float32),
    )(p3, st3, dis, b3)


def kernel(x, edge_index, W1, b1, gamma, beta, W2, b2, W3, b3):
    ei = edge_index.astype(jnp.int32)
    pad = E_PAD - N_EDGES
    src = jnp.concatenate([ei[0], jnp.zeros((pad,), jnp.int32)])
    dst = jnp.concatenate([ei[1], jnp.full((pad,), N_NODES, jnp.int32)])
    src2d = src.reshape(E_PAD // CHUNK, CHUNK)
    dst2d = dst.reshape(E_PAD // CHUNK, CHUNK)

    zeros1 = jnp.zeros((RPT,), jnp.float32)
    ones = jnp.ones((CHUNK,), jnp.float32)

    hist = _sc_degree(dst2d, ones, zeros1)
    m1a, m1b, st1, dis = _tc1(x, W1, hist)
    p1a = _sc_propagate(src2d, dst2d, m1a, 64, CH0, CH1)
    p1b = _sc_propagate(src2d, dst2d, m1b, 64, CH0, CH1)

    c = (1.0 + BN_EPS) ** -0.5
    gscale = (gamma * c).reshape(1, 128)
    bshift = (beta + b1 * gamma * c).reshape(1, 128)
    m2, st2 = _tc2(p1a, p1b, st1, dis, gscale, bshift, W2)
    p2 = _sc_propagate(src2d, dst2d, m2, 64, CH0, CH1)

    w3p = jnp.concatenate([W3, jnp.zeros((64, 14), jnp.float32)], axis=1)
    m3, st3 = _tc3(p2, st2, dis, b2.reshape(1, 64), w3p)
    p3 = _sc_propagate(src2d, dst2d, m3, 16, CH0, CH1)

    return _tc4(p3, st3, dis, b3.reshape(1, 2))
